# baseline (speedup 1.0000x reference)
import functools

import jax
import jax.numpy as jnp
from jax import lax
from jax.experimental import pallas as pl
from jax.experimental.pallas import tpu as pltpu
from jax.experimental.pallas import tpu_sc as plsc

NC = 2
NS = 16
NT = NC * NS
L = 16
EMB = 64
NTAGS = 16
N_LOOKUPS = 16384
NWORDS = 1000000

WBLK = 8192
CPAD = (NWORDS // WBLK + 1) * WBLK
SPAN = CPAD // NS
ZCH = SPAN // 8
PER_TILE = N_LOOKUPS // NT
CH = 128

TC_BLKS = 61
W0 = TC_BLKS * WBLK
UW = 512
RAG = (NWORDS - W0) % UW
N_UNITS = (NWORDS - W0 - RAG) // UW
UPT = 2 * ((N_UNITS + 2 * NT - 1) // (2 * NT))


def _sc_histogram(w2d):
    mesh = plsc.VectorSubcoreMesh(core_axis_name="c", subcore_axis_name="s")

    @functools.partial(
        pl.kernel,
        mesh=mesh,
        out_type=jax.ShapeDtypeStruct((NC, CPAD), jnp.float32),
        scratch_types=[
            pltpu.VMEM((PER_TILE // CH, CH), jnp.int32),
            pltpu.VMEM((CH,), jnp.float32),
            pltpu.VMEM((ZCH,), jnp.float32),
            pltpu.VMEM_SHARED((CPAD,), jnp.float32),
        ],
    )
    def hist(w_hbm, out_hbm, idx_v, ones_v, zb_v, csh):
        c = lax.axis_index("c")
        s = lax.axis_index("s")
        g = c * NS + s
        nrow = PER_TILE // CH
        pltpu.sync_copy(w_hbm.at[pl.ds(g * nrow, nrow)], idx_v)
        for kk in range(CH // L):
            ones_v[pl.ds(kk * L, L)] = jnp.ones((L,), jnp.float32)
        for kk in range(ZCH // L):
            zb_v[pl.ds(kk * L, L)] = jnp.zeros((L,), jnp.float32)
        for r in range(8):
            pltpu.sync_copy(zb_v, csh.at[pl.ds(s * SPAN + r * ZCH, ZCH)])
        plsc.subcore_barrier()
        for j in range(nrow):
            pltpu.sync_copy(ones_v, csh.at[idx_v.at[j]], add=True)
        plsc.subcore_barrier()
        pltpu.sync_copy(
            csh.at[pl.ds(s * SPAN, SPAN)],
            out_hbm.at[c, pl.ds(s * SPAN, SPAN)],
        )

    return hist(w2d)


def _sc_matvec(embT, C2):
    mesh = plsc.VectorSubcoreMesh(core_axis_name="c", subcore_axis_name="s")

    @functools.partial(
        pl.kernel,
        mesh=mesh,
        out_type=jax.ShapeDtypeStruct((NT, EMB, L), jnp.float32),
        scratch_types=[
            pltpu.VMEM((EMB, UW), jnp.float32),
            pltpu.VMEM((EMB, UW), jnp.float32),
            pltpu.VMEM((NC, UW), jnp.float32),
            pltpu.VMEM((NC, UW), jnp.float32),
            pltpu.VMEM((UW,), jnp.float32),
            pltpu.VMEM((EMB, L), jnp.float32),
            pltpu.SemaphoreType.DMA,
            pltpu.SemaphoreType.DMA,
        ],
    )
    def mv(e_hbm, c_hbm, out_hbm, db0, db1, cb0, cb1, cbs_v, acc_v,
           sem0, sem1):
        c = lax.axis_index("c")
        s = lax.axis_index("s")
        t = c * NS + s
        for f in range(EMB):
            acc_v[f, pl.ds(0, L)] = jnp.zeros((L,), jnp.float32)

        def unit_word(u):
            real = (t + NT * u) < N_UNITS
            return real, jnp.where(real, W0 + (t + NT * u) * UW, 0)

        def start(u, db, cb, sem):
            _, w = unit_word(u)
            ca = pltpu.async_copy(e_hbm.at[:, pl.ds(w, UW)], db, sem)
            cc = pltpu.async_copy(c_hbm.at[:, pl.ds(w, UW)], cb, sem)
            return ca, cc

        def drain(db, cb, sem):
            pltpu.make_async_copy(e_hbm.at[:, pl.ds(0, UW)], db, sem).wait()
            pltpu.make_async_copy(c_hbm.at[:, pl.ds(0, UW)], cb, sem).wait()

        def compute(u, db, cb, nk):
            real, _ = unit_word(u)
            rf = jnp.where(real, 1.0, 0.0).astype(jnp.float32)
            for kk in range(nk):
                cbs_v[pl.ds(kk * L, L)] = (
                    cb[0, pl.ds(kk * L, L)] + cb[1, pl.ds(kk * L, L)]
                ) * rf
            for fg in range(EMB // 8):
                def kb4(k4, accs, fg=fg):
                    res = list(accs)
                    for dk in range(4):
                        kpos = (k4 * 4 + dk) * L
                        ck = cbs_v[pl.ds(kpos, L)]
                        for r in range(8):
                            res[r] = res[r] + db[
                                fg * 8 + r, pl.ds(kpos, L)] * ck
                    return tuple(res)
                a = tuple(acc_v[fg * 8 + r, pl.ds(0, L)] for r in range(8))
                a = lax.fori_loop(0, nk // 4, kb4, a)
                for r in range(8):
                    acc_v[fg * 8 + r, pl.ds(0, L)] = a[r]

        start(0, db0, cb0, sem0)

        def body(j, carry):
            drain(db0, cb0, sem0)
            start(2 * j + 1, db1, cb1, sem1)
            compute(2 * j, db0, cb0, UW // L)
            drain(db1, cb1, sem1)

            @pl.when(j < UPT // 2 - 1)
            def _():
                start(2 * j + 2, db0, cb0, sem0)

            compute(2 * j + 1, db1, cb1, UW // L)
            return carry

        lax.fori_loop(0, UPT // 2, body, 0)

        pltpu.sync_copy(acc_v, out_hbm.at[t])

    return mv(embT, C2)


def _tc_matvec(embT, C2, tail, ct):
    def body(e_ref, c_ref, t_ref, ct_ref, o_ref, acc):
        i = pl.program_id(0)
        cb = c_ref[...]
        cbs = cb[0:1, :] + cb[1:2, :]
        s_step = jnp.sum(e_ref[...] * cbs, axis=1, keepdims=True)

        @pl.when(i == 0)
        def _():
            acc[...] = jnp.zeros_like(acc)

        acc[...] += s_step

        @pl.when(i == TC_BLKS - 1)
        def _():
            ctb = ct_ref[...]
            cts = ctb[0:1, :] + ctb[1:2, :]
            s_tail = jnp.sum(t_ref[...] * cts, axis=1, keepdims=True)
            o_ref[...] = acc[...] + s_tail

    return pl.pallas_call(
        body,
        grid=(TC_BLKS,),
        in_specs=[
            pl.BlockSpec((EMB, WBLK), lambda i: (0, i)),
            pl.BlockSpec((NC, WBLK), lambda i: (0, i)),
            pl.BlockSpec((EMB, RAG), lambda i: (0, 0)),
            pl.BlockSpec((NC, RAG), lambda i: (0, 0)),
        ],
        out_specs=pl.BlockSpec((EMB, 1), lambda i: (0, 0)),
        out_shape=jax.ShapeDtypeStruct((EMB, 1), jnp.float32),
        scratch_shapes=[pltpu.VMEM((EMB, 1), jnp.float32)],
    )(embT, C2, tail, ct)


def _tc_head(s_tc, P, WT, b2):
    def body(s_ref, p_ref, wt_ref, b_ref, o_ref):
        psum = jnp.sum(p_ref[...], axis=0)
        S = s_ref[...] + jnp.sum(psum, axis=1, keepdims=True)
        o_ref[...] = (
            jnp.dot(wt_ref[...], S, preferred_element_type=jnp.float32)
            + b_ref[...]
        )

    return pl.pallas_call(
        body,
        out_shape=jax.ShapeDtypeStruct((NTAGS, 1), jnp.float32),
    )(s_tc, P, WT, b2)


def kernel(words, embedding, W, b):
    w2d = words.astype(jnp.int32).reshape(CH, CH)
    C2 = _sc_histogram(w2d)
    embT = embedding.T
    tail = lax.slice(embT, (0, NWORDS - RAG), (EMB, NWORDS))
    ct = lax.slice(C2, (0, NWORDS - RAG), (NC, NWORDS))
    s_tc = _tc_matvec(embT, C2, tail, ct)
    P = _sc_matvec(embT, C2)
    out16 = _tc_head(s_tc, P, W.T, b.reshape(NTAGS, 1))
    return out16.reshape(1, NTAGS)


# speedup vs baseline: 1.0411x; 1.0411x over previous
import functools

import jax
import jax.numpy as jnp
from jax import lax
from jax.experimental import pallas as pl
from jax.experimental.pallas import tpu as pltpu
from jax.experimental.pallas import tpu_sc as plsc

NC = 2
NS = 16
NT = NC * NS
L = 16
EMB = 64
NTAGS = 16
N_LOOKUPS = 16384
NWORDS = 1000000

WBLK = 8192
CPAD = (NWORDS // WBLK + 1) * WBLK
SPAN = CPAD // NS
ZCH = SPAN // 8
PER_TILE = N_LOOKUPS // NT
CH = 128

TC_BLKS = 63
W0 = TC_BLKS * WBLK
UW = 512
RAG = (NWORDS - W0) % UW
N_UNITS = (NWORDS - W0 - RAG) // UW
UPT = 2 * ((N_UNITS + 2 * NT - 1) // (2 * NT))


def _sc_histogram(w2d):
    mesh = plsc.VectorSubcoreMesh(core_axis_name="c", subcore_axis_name="s")

    @functools.partial(
        pl.kernel,
        mesh=mesh,
        out_type=jax.ShapeDtypeStruct((NC, CPAD), jnp.float32),
        scratch_types=[
            pltpu.VMEM((PER_TILE // CH, CH), jnp.int32),
            pltpu.VMEM((CH,), jnp.float32),
            pltpu.VMEM((ZCH,), jnp.float32),
            pltpu.VMEM_SHARED((CPAD,), jnp.float32),
        ],
    )
    def hist(w_hbm, out_hbm, idx_v, ones_v, zb_v, csh):
        c = lax.axis_index("c")
        s = lax.axis_index("s")
        g = c * NS + s
        nrow = PER_TILE // CH
        pltpu.sync_copy(w_hbm.at[pl.ds(g * nrow, nrow)], idx_v)
        for kk in range(CH // L):
            ones_v[pl.ds(kk * L, L)] = jnp.ones((L,), jnp.float32)
        for kk in range(ZCH // L):
            zb_v[pl.ds(kk * L, L)] = jnp.zeros((L,), jnp.float32)
        for r in range(8):
            pltpu.sync_copy(zb_v, csh.at[pl.ds(s * SPAN + r * ZCH, ZCH)])
        plsc.subcore_barrier()
        for j in range(nrow):
            pltpu.sync_copy(ones_v, csh.at[idx_v.at[j]], add=True)
        plsc.subcore_barrier()
        pltpu.sync_copy(
            csh.at[pl.ds(s * SPAN, SPAN)],
            out_hbm.at[c, pl.ds(s * SPAN, SPAN)],
        )

    return hist(w2d)


def _sc_matvec(embT, C2):
    mesh = plsc.VectorSubcoreMesh(core_axis_name="c", subcore_axis_name="s")

    @functools.partial(
        pl.kernel,
        mesh=mesh,
        out_type=jax.ShapeDtypeStruct((NT, EMB, L), jnp.float32),
        scratch_types=[
            pltpu.VMEM((EMB, UW), jnp.float32),
            pltpu.VMEM((EMB, UW), jnp.float32),
            pltpu.VMEM((NC, UW), jnp.float32),
            pltpu.VMEM((NC, UW), jnp.float32),
            pltpu.VMEM((UW,), jnp.float32),
            pltpu.VMEM((EMB, L), jnp.float32),
            pltpu.SemaphoreType.DMA,
            pltpu.SemaphoreType.DMA,
        ],
    )
    def mv(e_hbm, c_hbm, out_hbm, db0, db1, cb0, cb1, cbs_v, acc_v,
           sem0, sem1):
        c = lax.axis_index("c")
        s = lax.axis_index("s")
        t = c * NS + s
        for f in range(EMB):
            acc_v[f, pl.ds(0, L)] = jnp.zeros((L,), jnp.float32)

        def unit_word(u):
            real = (t + NT * u) < N_UNITS
            return real, jnp.where(real, W0 + (t + NT * u) * UW, 0)

        def start(u, db, cb, sem):
            _, w = unit_word(u)
            ca = pltpu.async_copy(e_hbm.at[:, pl.ds(w, UW)], db, sem)
            cc = pltpu.async_copy(c_hbm.at[:, pl.ds(w, UW)], cb, sem)
            return ca, cc

        def drain(db, cb, sem):
            pltpu.make_async_copy(e_hbm.at[:, pl.ds(0, UW)], db, sem).wait()
            pltpu.make_async_copy(c_hbm.at[:, pl.ds(0, UW)], cb, sem).wait()

        def compute(u, db, cb, nk):
            real, _ = unit_word(u)
            rf = jnp.where(real, 1.0, 0.0).astype(jnp.float32)
            for kk in range(nk):
                cbs_v[pl.ds(kk * L, L)] = (
                    cb[0, pl.ds(kk * L, L)] + cb[1, pl.ds(kk * L, L)]
                ) * rf
            for fg in range(EMB // 8):
                def kb4(k4, accs, fg=fg):
                    res = list(accs)
                    for dk in range(4):
                        kpos = (k4 * 4 + dk) * L
                        ck = cbs_v[pl.ds(kpos, L)]
                        for r in range(8):
                            res[r] = res[r] + db[
                                fg * 8 + r, pl.ds(kpos, L)] * ck
                    return tuple(res)
                a = tuple(acc_v[fg * 8 + r, pl.ds(0, L)] for r in range(8))
                a = lax.fori_loop(0, nk // 4, kb4, a)
                for r in range(8):
                    acc_v[fg * 8 + r, pl.ds(0, L)] = a[r]

        start(0, db0, cb0, sem0)

        def body(j, carry):
            drain(db0, cb0, sem0)
            start(2 * j + 1, db1, cb1, sem1)
            compute(2 * j, db0, cb0, UW // L)
            drain(db1, cb1, sem1)

            @pl.when(j < UPT // 2 - 1)
            def _():
                start(2 * j + 2, db0, cb0, sem0)

            compute(2 * j + 1, db1, cb1, UW // L)
            return carry

        lax.fori_loop(0, UPT // 2, body, 0)

        pltpu.sync_copy(acc_v, out_hbm.at[t])

    return mv(embT, C2)


def _tc_matvec(embT, C2, tail, ct):
    def body(e_ref, c_ref, t_ref, ct_ref, o_ref, acc):
        i = pl.program_id(0)
        cb = c_ref[...]
        cbs = cb[0:1, :] + cb[1:2, :]
        s_step = jnp.sum(e_ref[...] * cbs, axis=1, keepdims=True)

        @pl.when(i == 0)
        def _():
            acc[...] = jnp.zeros_like(acc)

        acc[...] += s_step

        @pl.when(i == TC_BLKS - 1)
        def _():
            ctb = ct_ref[...]
            cts = ctb[0:1, :] + ctb[1:2, :]
            s_tail = jnp.sum(t_ref[...] * cts, axis=1, keepdims=True)
            o_ref[...] = acc[...] + s_tail

    return pl.pallas_call(
        body,
        grid=(TC_BLKS,),
        in_specs=[
            pl.BlockSpec((EMB, WBLK), lambda i: (0, i)),
            pl.BlockSpec((NC, WBLK), lambda i: (0, i)),
            pl.BlockSpec((EMB, RAG), lambda i: (0, 0)),
            pl.BlockSpec((NC, RAG), lambda i: (0, 0)),
        ],
        out_specs=pl.BlockSpec((EMB, 1), lambda i: (0, 0)),
        out_shape=jax.ShapeDtypeStruct((EMB, 1), jnp.float32),
        scratch_shapes=[pltpu.VMEM((EMB, 1), jnp.float32)],
    )(embT, C2, tail, ct)


def _tc_head(s_tc, P, WT, b2):
    def body(s_ref, p_ref, wt_ref, b_ref, o_ref):
        psum = jnp.sum(p_ref[...], axis=0)
        S = s_ref[...] + jnp.sum(psum, axis=1, keepdims=True)
        o_ref[...] = (
            jnp.dot(wt_ref[...], S, preferred_element_type=jnp.float32)
            + b_ref[...]
        )

    return pl.pallas_call(
        body,
        out_shape=jax.ShapeDtypeStruct((NTAGS, 1), jnp.float32),
    )(s_tc, P, WT, b2)


def kernel(words, embedding, W, b):
    w2d = words.astype(jnp.int32).reshape(CH, CH)
    C2 = _sc_histogram(w2d)
    embT = embedding.T
    tail = lax.slice(embT, (0, NWORDS - RAG), (EMB, NWORDS))
    ct = lax.slice(C2, (0, NWORDS - RAG), (NC, NWORDS))
    s_tc = _tc_matvec(embT, C2, tail, ct)
    P = _sc_matvec(embT, C2)
    out16 = _tc_head(s_tc, P, W.T, b.reshape(NTAGS, 1))
    return out16.reshape(1, NTAGS)
